# baseline (speedup 1.0000x reference)
import jax
import jax.numpy as jnp
from jax.experimental import pallas as pl


def _body(x_ref, tmpl_ref, o_ref):
    bb = x_ref.shape[0]
    nt = x_ref.shape[1] // 128
    xb = x_ref[...]
    tiles = []
    for t in range(nt):
        xt = xb[:, 128 * t:128 * (t + 1)]
        tiles.append(xt[:, :64])
        tiles.append(jnp.broadcast_to(
            tmpl_ref[:, 256 * t + 64:256 * t + 128], (bb, 64)))
        tiles.append(xt[:, 64:])
        tiles.append(jnp.broadcast_to(
            tmpl_ref[:, 256 * t + 192:256 * t + 256], (bb, 64)))
    o_ref[...] = jnp.concatenate(tiles, axis=1)


def kernel(x, position_table):
    B, L, D = x.shape
    pos = position_table[:L]
    tmpl = jnp.concatenate(
        [jnp.zeros((L, D), pos.dtype), pos], axis=-1).reshape(1, L * 2 * D)
    x2 = x.reshape(B, L * D)
    BB = 128
    NC = 4
    XW = L * D // NC
    OW = L * 2 * D // NC
    out = pl.pallas_call(
        _body,
        grid=(B // BB, NC),
        in_specs=[
            pl.BlockSpec((BB, XW), lambda i, j: (i, j)),
            pl.BlockSpec((1, OW), lambda i, j: (0, j)),
        ],
        out_specs=pl.BlockSpec((BB, OW), lambda i, j: (i, j)),
        out_shape=jax.ShapeDtypeStruct((B, L * 2 * D), x.dtype),
    )(x2, tmpl)
    return out.reshape(B, L, 2 * D)


# speedup vs baseline: 1.0317x; 1.0317x over previous
import jax
import jax.numpy as jnp
from jax.experimental import pallas as pl
from jax.experimental.pallas import tpu as pltpu

_BB = 128


def _interleave(x_buf, tmpl_ref, o_buf):
    bb = x_buf.shape[0]
    nt = x_buf.shape[1] // 128
    xb = x_buf[...]
    tiles = []
    for t in range(nt):
        xt = xb[:, 128 * t:128 * (t + 1)]
        tiles.append(xt[:, :64])
        tiles.append(jnp.broadcast_to(
            tmpl_ref[:, 256 * t + 64:256 * t + 128], (bb, 64)))
        tiles.append(xt[:, 64:])
        tiles.append(jnp.broadcast_to(
            tmpl_ref[:, 256 * t + 192:256 * t + 256], (bb, 64)))
    o_buf[...] = jnp.concatenate(tiles, axis=1)


def _body(x_hbm, tmpl_ref, o_hbm,
          in0, in1, out0, out1, in_sems, out_sems):
    i = pl.program_id(0)
    n = pl.num_programs(0)

    def in_copy(idx, buf, sem):
        return pltpu.make_async_copy(
            x_hbm.at[pl.ds(idx * _BB, _BB), :], buf, sem)

    def out_copy(idx, buf, sem):
        return pltpu.make_async_copy(
            buf, o_hbm.at[pl.ds(idx * _BB, _BB), :], sem)

    def phase(cur_in, oth_in, cur_out, oth_out,
              cur_is, oth_is, cur_os, oth_os):
        @pl.when(i == 0)
        def _():
            in_copy(i, cur_in, cur_is).start()

        @pl.when(i + 1 < n)
        def _():
            in_copy(i + 1, oth_in, oth_is).start()

        in_copy(i, cur_in, cur_is).wait()

        @pl.when(i >= 2)
        def _():
            out_copy(i - 2, cur_out, cur_os).wait()

        _interleave(cur_in, tmpl_ref, cur_out)
        out_copy(i, cur_out, cur_os).start()

        @pl.when(i == n - 1)
        def _():
            out_copy(i, cur_out, cur_os).wait()
            out_copy(i - 1, oth_out, oth_os).wait()

    @pl.when(i % 2 == 0)
    def _():
        phase(in0, in1, out0, out1,
              in_sems.at[0], in_sems.at[1], out_sems.at[0], out_sems.at[1])

    @pl.when(i % 2 == 1)
    def _():
        phase(in1, in0, out1, out0,
              in_sems.at[1], in_sems.at[0], out_sems.at[1], out_sems.at[0])


def kernel(x, position_table):
    B, L, D = x.shape
    pos = position_table[:L]
    tmpl = jnp.concatenate(
        [jnp.zeros((L, D), pos.dtype), pos], axis=-1).reshape(1, L * 2 * D)
    x2 = x.reshape(B, L * D)
    XW = L * D
    OW = L * 2 * D
    out = pl.pallas_call(
        _body,
        grid=(B // _BB,),
        in_specs=[
            pl.BlockSpec(memory_space=pl.ANY),
            pl.BlockSpec((1, OW), lambda i: (0, 0)),
        ],
        out_specs=pl.BlockSpec(memory_space=pl.ANY),
        out_shape=jax.ShapeDtypeStruct((B, OW), x.dtype),
        scratch_shapes=[
            pltpu.VMEM((_BB, XW), jnp.float32),
            pltpu.VMEM((_BB, XW), jnp.float32),
            pltpu.VMEM((_BB, OW), jnp.float32),
            pltpu.VMEM((_BB, OW), jnp.float32),
            pltpu.SemaphoreType.DMA((2,)),
            pltpu.SemaphoreType.DMA((2,)),
        ],
    )(x2, tmpl)
    return out.reshape(B, L, 2 * D)
